# baseline (speedup 1.0000x reference)
import jax
import jax.numpy as jnp
from jax import lax
from jax.experimental import pallas as pl
from jax.experimental.pallas import tpu as pltpu
from jax.experimental.pallas import tpu_sc as plsc

EPS = 1e-6
B = 8
IN_SIZE = 4096
OUT_SIZE = 1024
K = 32768
NC = 2
NS = 16
L = 16
NW = NC * NS
KPT = K // NW
NVEC = KPT // L
UNROLL = 1
OUT_FLAT = B * OUT_SIZE
RED = OUT_FLAT // NS
RPR = OUT_SIZE // RED


def _sc_body(x_hbm, means_hbm, sigmas_hbm, val_hbm, out_hbm,
             x_v, acc_v, mu_v, sg_v, val_v, stage_v, red_v,
             shared, dma_sem):
    c = lax.axis_index("c")
    s = lax.axis_index("s")
    wid = c * NS + s
    kbase = wid * KPT

    copies = [
        pltpu.async_copy(x_hbm.at[b], x_v.at[pl.ds(b * IN_SIZE, IN_SIZE)],
                         dma_sem)
        for b in range(B)
    ]
    copies.append(pltpu.async_copy(means_hbm.at[pl.ds(2 * kbase, 2 * KPT)],
                                   mu_v, dma_sem))
    copies.append(pltpu.async_copy(sigmas_hbm.at[pl.ds(2 * kbase, 2 * KPT)],
                                   sg_v, dma_sem))
    copies.append(pltpu.async_copy(val_hbm.at[pl.ds(kbase, KPT)], val_v,
                                   dma_sem))

    zeros = jnp.zeros((L,), jnp.float32)

    @plsc.parallel_loop(0, OUT_FLAT, L, unroll=4)
    def zero_body(o):
        acc_v[pl.ds(o, L)] = zeros

    for cp in copies:
        cp.wait()

    lanes2 = lax.iota(jnp.int32, L) * 2

    def do_chunk(base):
        rows2 = 2 * base + lanes2
        rows2b = rows2 + 1
        m0 = plsc.load_gather(mu_v, [rows2])
        m1 = plsc.load_gather(mu_v, [rows2b])
        s0 = plsc.load_gather(sg_v, [rows2])
        s1 = plsc.load_gather(sg_v, [rows2b])
        vv = val_v[pl.dslice(base, L)]

        fi0 = m0.astype(jnp.int32)
        fv0 = fi0.astype(jnp.float32)
        ci0 = jnp.where(fv0 == m0, fi0, fi0 + 1)
        cv0 = ci0.astype(jnp.float32)
        fi1 = m1.astype(jnp.int32)
        fv1 = fi1.astype(jnp.float32)
        ci1 = jnp.where(fv1 == m1, fi1, fi1 + 1)
        cv1 = ci1.astype(jnp.float32)

        w0 = 1.0 / (EPS + s0)
        w1 = 1.0 / (EPS + s1)
        d = fv0 - m0
        e00 = d * d * w0
        d = cv0 - m0
        e01 = d * d * w0
        d = fv1 - m1
        e10 = d * d * w1
        d = cv1 - m1
        e11 = d * d * w1

        p0 = jnp.exp(-0.5 * (e00 + e10))
        p1 = jnp.exp(-0.5 * (e00 + e11))
        p2 = jnp.exp(-0.5 * (e01 + e10))
        p3 = jnp.exp(-0.5 * (e01 + e11))
        vn = vv / (p0 + p1 + p2 + p3 + EPS)
        v0 = p0 * vn
        v1 = p1 * vn
        v2 = p2 * vn
        v3 = p3 * vn

        of = jnp.minimum(jnp.maximum(fi0, 0), OUT_SIZE - 1)
        oc = jnp.minimum(jnp.maximum(ci0, 0), OUT_SIZE - 1)
        gf = jnp.minimum(jnp.maximum(fi1, 0), IN_SIZE - 1)
        gc = jnp.minimum(jnp.maximum(ci1, 0), IN_SIZE - 1)
        for b in range(B):
            xf = plsc.load_gather(x_v, [gf + b * IN_SIZE])
            xc = plsc.load_gather(x_v, [gc + b * IN_SIZE])
            plsc.addupdate_scatter(acc_v, [of + b * OUT_SIZE],
                                   v0 * xf + v1 * xc)
            plsc.addupdate_scatter(acc_v, [oc + b * OUT_SIZE],
                                   v2 * xf + v3 * xc)

    @plsc.parallel_loop(0, KPT, L, unroll=UNROLL)
    def chunk_loop(base):
        do_chunk(base)

    pltpu.sync_copy(acc_v, shared.at[s])
    plsc.subcore_barrier()

    rbase = s * RED
    pltpu.sync_copy(shared.at[:, pl.ds(rbase, RED)], stage_v)

    @plsc.parallel_loop(0, RED, L, unroll=2)
    def red_body(o):
        acc = stage_v[0, pl.ds(o, L)]
        for t in range(1, NS):
            acc = acc + stage_v[t, pl.ds(o, L)]
        red_v[pl.ds(o, L)] = acc

    pltpu.sync_copy(red_v,
                    out_hbm.at[c, s // RPR, pl.ds((s % RPR) * RED, RED)])


def _combine_body(p_ref, b_ref, o_ref):
    o_ref[...] = p_ref[0] + p_ref[1] + b_ref[...]


@jax.jit
def _run(input, means, sigmas, values, bias):
    mesh = plsc.VectorSubcoreMesh(core_axis_name="c", subcore_axis_name="s",
                                  num_cores=NC, num_subcores=NS)
    partials = pl.kernel(
        _sc_body,
        out_type=jax.ShapeDtypeStruct((NC, B, OUT_SIZE), jnp.float32),
        mesh=mesh,
        scratch_types=[
            pltpu.VMEM((B * IN_SIZE,), jnp.float32),
            pltpu.VMEM((OUT_FLAT,), jnp.float32),
            pltpu.VMEM((2 * KPT,), jnp.float32),
            pltpu.VMEM((2 * KPT,), jnp.float32),
            pltpu.VMEM((KPT,), jnp.float32),
            pltpu.VMEM((NS, RED), jnp.float32),
            pltpu.VMEM((RED,), jnp.float32),
            pltpu.VMEM_SHARED((NS, OUT_FLAT), jnp.float32),
            pltpu.SemaphoreType.DMA,
        ],
        compiler_params=pltpu.CompilerParams(needs_layout_passes=False),
    )(input, means.reshape(-1), sigmas.reshape(-1), values)
    return pl.pallas_call(
        _combine_body,
        out_shape=jax.ShapeDtypeStruct((B, OUT_SIZE), jnp.float32),
    )(partials, bias.reshape(1, OUT_SIZE))


def kernel(input, means, sigmas, values, bias):
    return _run(input, means, sigmas, values, bias)


# speedup vs baseline: 2.2770x; 2.2770x over previous
import jax
import jax.numpy as jnp
from jax import lax
from jax.experimental import pallas as pl
from jax.experimental.pallas import tpu as pltpu
from jax.experimental.pallas import tpu_sc as plsc

EPS = 1e-6
B = 8
IN_SIZE = 4096
OUT_SIZE = 1024
K = 32768
NC = 2
NS = 16
L = 16
NW = NC * NS
KPT = K // NW
NVEC = KPT // L
UNROLL = 1
OUT_FLAT = B * OUT_SIZE
RED = OUT_FLAT // NS
RPR = OUT_SIZE // RED


def _sc_body(x_hbm, m0_hbm, m1_hbm, s0_hbm, s1_hbm, val_hbm, out_hbm,
             x_v, acc_v, m0_v, m1_v, s0_v, s1_v, val_v, stage_v, red_v,
             shared, dma_sem):
    c = lax.axis_index("c")
    s = lax.axis_index("s")
    wid = c * NS + s
    kbase = wid * KPT

    copies = [
        pltpu.async_copy(x_hbm.at[b], x_v.at[pl.ds(b * IN_SIZE, IN_SIZE)],
                         dma_sem)
        for b in range(B)
    ]
    copies.append(pltpu.async_copy(m0_hbm.at[pl.ds(kbase, KPT)], m0_v,
                                   dma_sem))
    copies.append(pltpu.async_copy(m1_hbm.at[pl.ds(kbase, KPT)], m1_v,
                                   dma_sem))
    copies.append(pltpu.async_copy(s0_hbm.at[pl.ds(kbase, KPT)], s0_v,
                                   dma_sem))
    copies.append(pltpu.async_copy(s1_hbm.at[pl.ds(kbase, KPT)], s1_v,
                                   dma_sem))
    copies.append(pltpu.async_copy(val_hbm.at[pl.ds(kbase, KPT)], val_v,
                                   dma_sem))

    zeros = jnp.zeros((L,), jnp.float32)

    @plsc.parallel_loop(0, OUT_FLAT, L, unroll=4)
    def zero_body(o):
        acc_v[pl.ds(o, L)] = zeros

    for cp in copies:
        cp.wait()

    def do_chunk(base):
        m0 = m0_v[pl.dslice(base, L)]
        m1 = m1_v[pl.dslice(base, L)]
        s0 = s0_v[pl.dslice(base, L)]
        s1 = s1_v[pl.dslice(base, L)]
        vv = val_v[pl.dslice(base, L)]

        fi0 = m0.astype(jnp.int32)
        fv0 = fi0.astype(jnp.float32)
        ci0 = jnp.where(fv0 == m0, fi0, fi0 + 1)
        cv0 = ci0.astype(jnp.float32)
        fi1 = m1.astype(jnp.int32)
        fv1 = fi1.astype(jnp.float32)
        ci1 = jnp.where(fv1 == m1, fi1, fi1 + 1)
        cv1 = ci1.astype(jnp.float32)

        w0 = 1.0 / (EPS + s0)
        w1 = 1.0 / (EPS + s1)
        d = fv0 - m0
        e00 = d * d * w0
        d = cv0 - m0
        e01 = d * d * w0
        d = fv1 - m1
        e10 = d * d * w1
        d = cv1 - m1
        e11 = d * d * w1

        p0 = jnp.exp(-0.5 * (e00 + e10))
        p1 = jnp.exp(-0.5 * (e00 + e11))
        p2 = jnp.exp(-0.5 * (e01 + e10))
        p3 = jnp.exp(-0.5 * (e01 + e11))
        vn = vv / (p0 + p1 + p2 + p3 + EPS)
        v0 = p0 * vn
        v1 = p1 * vn
        v2 = p2 * vn
        v3 = p3 * vn

        of = jnp.minimum(jnp.maximum(fi0, 0), OUT_SIZE - 1)
        oc = jnp.minimum(jnp.maximum(ci0, 0), OUT_SIZE - 1)
        gf = jnp.minimum(jnp.maximum(fi1, 0), IN_SIZE - 1)
        gc = jnp.minimum(jnp.maximum(ci1, 0), IN_SIZE - 1)
        for b in range(B):
            xf = plsc.load_gather(x_v, [gf + b * IN_SIZE])
            xc = plsc.load_gather(x_v, [gc + b * IN_SIZE])
            plsc.addupdate_scatter(acc_v, [of + b * OUT_SIZE],
                                   v0 * xf + v1 * xc)
            plsc.addupdate_scatter(acc_v, [oc + b * OUT_SIZE],
                                   v2 * xf + v3 * xc)

    @plsc.parallel_loop(0, KPT, L, unroll=UNROLL)
    def chunk_loop(base):
        do_chunk(base)

    pltpu.sync_copy(acc_v, shared.at[s])
    plsc.subcore_barrier()

    rbase = s * RED
    pltpu.sync_copy(shared.at[:, pl.ds(rbase, RED)], stage_v)

    @plsc.parallel_loop(0, RED, L, unroll=2)
    def red_body(o):
        acc = stage_v[0, pl.ds(o, L)]
        for t in range(1, NS):
            acc = acc + stage_v[t, pl.ds(o, L)]
        red_v[pl.ds(o, L)] = acc

    pltpu.sync_copy(red_v,
                    out_hbm.at[c, s // RPR, pl.ds((s % RPR) * RED, RED)])


def _combine_body(p_ref, b_ref, o_ref):
    o_ref[...] = p_ref[0] + p_ref[1] + b_ref[...]


@jax.jit
def _run(input, means, sigmas, values, bias):
    mesh = plsc.VectorSubcoreMesh(core_axis_name="c", subcore_axis_name="s",
                                  num_cores=NC, num_subcores=NS)
    partials = pl.kernel(
        _sc_body,
        out_type=jax.ShapeDtypeStruct((NC, B, OUT_SIZE), jnp.float32),
        mesh=mesh,
        scratch_types=[
            pltpu.VMEM((B * IN_SIZE,), jnp.float32),
            pltpu.VMEM((OUT_FLAT,), jnp.float32),
            pltpu.VMEM((KPT,), jnp.float32),
            pltpu.VMEM((KPT,), jnp.float32),
            pltpu.VMEM((KPT,), jnp.float32),
            pltpu.VMEM((KPT,), jnp.float32),
            pltpu.VMEM((KPT,), jnp.float32),
            pltpu.VMEM((NS, RED), jnp.float32),
            pltpu.VMEM((RED,), jnp.float32),
            pltpu.VMEM_SHARED((NS, OUT_FLAT), jnp.float32),
            pltpu.SemaphoreType.DMA,
        ],
        compiler_params=pltpu.CompilerParams(needs_layout_passes=False),
    )(input, means[:, 0], means[:, 1], sigmas[:, 0], sigmas[:, 1], values)
    return pl.pallas_call(
        _combine_body,
        out_shape=jax.ShapeDtypeStruct((B, OUT_SIZE), jnp.float32),
    )(partials, bias.reshape(1, OUT_SIZE))


def kernel(input, means, sigmas, values, bias):
    return _run(input, means, sigmas, values, bias)
